# baseline (speedup 1.0000x reference)
import functools

import jax
import jax.numpy as jnp
from jax import lax
from jax.experimental import pallas as pl
from jax.experimental.pallas import tpu as pltpu
from jax.experimental.pallas import tpu_sc as plsc

VOCAB = 5977
INV_SLOTS = 55
EDIM = 32
HDIM = 128
BATCH = 16384

NC = 2
NS = 16
NW = NC * NS

SLOT_PAD = 56
CGROUPS = SLOT_PAD * EDIM // 128
N_ROWS = BATCH * SLOT_PAD
ROWS_PER_W = N_ROWS // NW
CHUNK = 2048
N_CHUNKS = ROWS_PER_W // CHUNK


def _gather_body(idx_hbm, emb_hbm, out_hbm, idx_v, rows_v, sem):
    wid = lax.axis_index("s") * NC + lax.axis_index("c")
    base = wid * ROWS_PER_W
    for k in range(N_CHUNKS):
        off = base + k * CHUNK
        pltpu.sync_copy(idx_hbm.at[pl.ds(off, CHUNK)], idx_v)
        pltpu.async_copy(emb_hbm.at[idx_v], rows_v, sem).wait()
        pltpu.sync_copy(rows_v, out_hbm.at[pl.ds(off, CHUNK)])


@functools.cache
def _sc_gather():
    return pl.kernel(
        _gather_body,
        out_type=jax.ShapeDtypeStruct((N_ROWS, EDIM), jnp.float32),
        mesh=plsc.VectorSubcoreMesh(core_axis_name="c", subcore_axis_name="s"),
        scratch_types=[
            pltpu.VMEM((CHUNK,), jnp.int32),
            pltpu.VMEM((CHUNK, EDIM), jnp.float32),
            pltpu.SemaphoreType.DMA,
        ],
        compiler_params=pltpu.CompilerParams(use_tc_tiling_on_sc=False),
    )


def _mlp_body(x_ref, w1_ref, b1_ref, g_ref, bt_ref, w2_ref, b2_ref, o_ref):
    h = b1_ref[...]
    for c in range(CGROUPS):
        xc = x_ref[c].astype(jnp.bfloat16)
        h = h + jnp.dot(xc, w1_ref[c], preferred_element_type=jnp.float32)
    mean = jnp.mean(h, axis=1, keepdims=True)
    var = jnp.mean((h - mean) ** 2, axis=1, keepdims=True)
    ln = (h - mean) * lax.rsqrt(var + 1e-5) * g_ref[...] + bt_ref[...]
    a = jnp.where(ln > 0, ln, jnp.exp(ln) - 1.0)
    o_ref[...] = jnp.dot(a, w2_ref[...], preferred_element_type=jnp.float32) + b2_ref[...]


def _mlp(x3, W1g, b1, gamma, beta, W2, b2, block_b=1024):
    grid = (BATCH // block_b,)
    return pl.pallas_call(
        _mlp_body,
        grid=grid,
        in_specs=[
            pl.BlockSpec((CGROUPS, block_b, 128), lambda i: (0, i, 0)),
            pl.BlockSpec((CGROUPS, 128, HDIM), lambda i: (0, 0, 0)),
            pl.BlockSpec((1, HDIM), lambda i: (0, 0)),
            pl.BlockSpec((1, HDIM), lambda i: (0, 0)),
            pl.BlockSpec((1, HDIM), lambda i: (0, 0)),
            pl.BlockSpec((HDIM, HDIM), lambda i: (0, 0)),
            pl.BlockSpec((1, HDIM), lambda i: (0, 0)),
        ],
        out_specs=pl.BlockSpec((block_b, HDIM), lambda i: (i, 0)),
        out_shape=jax.ShapeDtypeStruct((BATCH, HDIM), jnp.float32),
        compiler_params=pltpu.CompilerParams(
            dimension_semantics=("arbitrary",),
        ),
    )(x3, W1g, b1, gamma, beta, W2, b2)


def kernel(inv_glyphs, emb, W1, b1, gamma, beta, W2, b2):
    idx = jnp.pad(inv_glyphs.astype(jnp.int32), ((0, 0), (0, SLOT_PAD - INV_SLOTS)))
    idx = idx.reshape(BATCH, CGROUPS, 4).transpose(1, 0, 2).reshape(-1)
    rows = _sc_gather()(idx, emb)
    x3 = rows.reshape(CGROUPS, BATCH, 128)
    W1g = jnp.pad(W1, ((0, SLOT_PAD * EDIM - W1.shape[0]), (0, 0)))
    W1g = W1g.reshape(CGROUPS, 128, HDIM).astype(jnp.bfloat16)
    return _mlp(x3, W1g, b1.reshape(1, HDIM), gamma.reshape(1, HDIM),
                beta.reshape(1, HDIM), W2, b2.reshape(1, HDIM))


# speedup vs baseline: 2.6101x; 2.6101x over previous
import functools

import jax
import jax.numpy as jnp
from jax import lax
from jax.experimental import pallas as pl
from jax.experimental.pallas import tpu as pltpu
from jax.experimental.pallas import tpu_sc as plsc

VOCAB = 5977
INV_SLOTS = 55
EDIM = 32
HDIM = 128
BATCH = 16384

NC = 2
NS = 16
NW = NC * NS

SLOT_PAD = 56
CGROUPS = SLOT_PAD * EDIM // 128
N_ROWS = BATCH * SLOT_PAD
ROWS_PER_W = N_ROWS // NW
CB = 512
CHUNK = CB * 4
BCHUNKS = BATCH // CB
N_CHUNKS = CGROUPS * BCHUNKS // NW


def _gather_body(idx_hbm, emb_hbm, out_hbm, idx_v, rows_v, sem):
    wid = lax.axis_index("s") * NC + lax.axis_index("c")
    for k in range(N_CHUNKS):
        t = wid * N_CHUNKS + k
        c = t // BCHUNKS
        b0 = (t % BCHUNKS) * CB
        pltpu.sync_copy(idx_hbm.at[pl.ds(t * CHUNK, CHUNK)], idx_v)
        pltpu.async_copy(emb_hbm.at[idx_v], rows_v, sem).wait()
        for j in range(4):
            pltpu.sync_copy(
                rows_v.at[pl.ds(j * CB, CB), :],
                out_hbm.at[c, pl.ds(b0, CB), pl.ds(32 * j, 32)])


@functools.cache
def _sc_gather():
    return pl.kernel(
        _gather_body,
        out_type=jax.ShapeDtypeStruct((CGROUPS, BATCH, 128), jnp.float32),
        mesh=plsc.VectorSubcoreMesh(core_axis_name="c", subcore_axis_name="s"),
        scratch_types=[
            pltpu.VMEM((CHUNK,), jnp.int32),
            pltpu.VMEM((CHUNK, EDIM), jnp.float32),
            pltpu.SemaphoreType.DMA,
        ],
        compiler_params=pltpu.CompilerParams(use_tc_tiling_on_sc=False),
    )


def _mlp_body(x_ref, w1_ref, b1_ref, g_ref, bt_ref, w2_ref, b2_ref, o_ref):
    h = b1_ref[...]
    for c in range(CGROUPS):
        xc = x_ref[c].astype(jnp.bfloat16)
        h = h + jnp.dot(xc, w1_ref[c], preferred_element_type=jnp.float32)
    mean = jnp.mean(h, axis=1, keepdims=True)
    var = jnp.mean((h - mean) ** 2, axis=1, keepdims=True)
    ln = (h - mean) * lax.rsqrt(var + 1e-5) * g_ref[...] + bt_ref[...]
    a = jnp.where(ln > 0, ln, jnp.exp(ln) - 1.0)
    o_ref[...] = jnp.dot(a, w2_ref[...], preferred_element_type=jnp.float32) + b2_ref[...]


def _mlp(x3, W1g, b1, gamma, beta, W2, b2, block_b=1024):
    grid = (BATCH // block_b,)
    return pl.pallas_call(
        _mlp_body,
        grid=grid,
        in_specs=[
            pl.BlockSpec((CGROUPS, block_b, 128), lambda i: (0, i, 0)),
            pl.BlockSpec((CGROUPS, 128, HDIM), lambda i: (0, 0, 0)),
            pl.BlockSpec((1, HDIM), lambda i: (0, 0)),
            pl.BlockSpec((1, HDIM), lambda i: (0, 0)),
            pl.BlockSpec((1, HDIM), lambda i: (0, 0)),
            pl.BlockSpec((HDIM, HDIM), lambda i: (0, 0)),
            pl.BlockSpec((1, HDIM), lambda i: (0, 0)),
        ],
        out_specs=pl.BlockSpec((block_b, HDIM), lambda i: (i, 0)),
        out_shape=jax.ShapeDtypeStruct((BATCH, HDIM), jnp.float32),
        compiler_params=pltpu.CompilerParams(
            dimension_semantics=("arbitrary",),
        ),
    )(x3, W1g, b1, gamma, beta, W2, b2)


def kernel(inv_glyphs, emb, W1, b1, gamma, beta, W2, b2):
    pad_col = (jnp.arange(BATCH, dtype=jnp.int32) % VOCAB)[:, None]
    idx = jnp.concatenate([inv_glyphs.astype(jnp.int32), pad_col], axis=1)
    idx = (idx.reshape(BCHUNKS, CB, CGROUPS, 4)
           .transpose(2, 0, 3, 1).reshape(-1))
    x3 = _sc_gather()(idx, emb)
    W1g = jnp.pad(W1, ((0, SLOT_PAD * EDIM - W1.shape[0]), (0, 0)))
    W1g = W1g.reshape(CGROUPS, 128, HDIM).astype(jnp.bfloat16)
    return _mlp(x3, W1g, b1.reshape(1, HDIM), gamma.reshape(1, HDIM),
                beta.reshape(1, HDIM), W2, b2.reshape(1, HDIM))
